# Initial kernel scaffold; baseline (speedup 1.0000x reference)
import jax
import jax.numpy as jnp
from jax.experimental import pallas as pl


def kernel(pc0_points, pc1_points, coarse_flow, W1, b1, W2, b2, W3, b3):
    raise NotImplementedError("write your pallas kernel here")


# speedup vs baseline: 4.7882x; 4.7882x over previous
import functools

import jax
import jax.numpy as jnp
from jax.experimental import pallas as pl

N = 8192
M = 8192
K = 16
R2 = 1.0
HIDDEN = 64
BLK = 256


def _knn_mlp_kernel(pc0_ref, coarse_ref, pc1_ref, pc1t_ref,
                    w1_ref, b1_ref, w2_ref, b2_ref, w3_ref, b3_ref,
                    fine_ref, warped_ref):
    pc0 = pc0_ref[...]
    coarse = coarse_ref[...]
    target = pc0 + coarse
    pc1 = pc1_ref[...]
    pc1t = pc1t_ref[...]

    tn = jnp.sum(target * target, axis=1, keepdims=True)
    pn = jnp.sum(pc1t * pc1t, axis=0, keepdims=True)
    tpad = jnp.concatenate(
        [target, jnp.zeros((BLK, 5), jnp.float32)], axis=1)
    dot = jnp.dot(tpad.astype(jnp.bfloat16), pc1t.astype(jnp.bfloat16),
                  preferred_element_type=jnp.float32)
    d2 = tn + pn - 2.0 * dot

    col = jax.lax.broadcasted_iota(jnp.int32, (BLK, M), 1)
    big_i = jnp.int32(M)

    feats_rows = []
    valid_rows = []
    any_valid = jnp.zeros((BLK, 1), dtype=jnp.bool_)
    for _ in range(K):
        m = jnp.min(d2, axis=1, keepdims=True)
        is_min = d2 == m
        idx = jnp.min(jnp.where(is_min, col, big_i), axis=1, keepdims=True)
        onehot = (col == idx).astype(jnp.float32)
        pts = jnp.dot(onehot, pc1, preferred_element_type=jnp.float32)
        valid = jnp.sqrt(jnp.maximum(m, 1e-12)) < 1.0
        any_valid = jnp.logical_or(any_valid, valid)
        local = pts - target
        feats_rows.append(jnp.concatenate([local, pc0], axis=1))
        valid_rows.append(valid)
        d2 = jnp.where(col == idx, jnp.float32(3e38), d2)

    feats = jnp.concatenate(feats_rows, axis=0)
    vmask = jnp.concatenate(valid_rows, axis=0)

    h = jnp.dot(feats, w1_ref[...], preferred_element_type=jnp.float32)
    h = jnp.maximum(h + b1_ref[...], 0.0)
    h = jnp.dot(h, w2_ref[...], preferred_element_type=jnp.float32)
    h = jnp.maximum(h + b2_ref[...], 0.0)
    h = jnp.where(vmask, h, jnp.float32(-1e9))
    pooled = jnp.max(h.reshape(K, BLK, HIDDEN), axis=0)
    pooled = jnp.where(any_valid, pooled, 0.0)

    residual = jnp.dot(pooled, w3_ref[...], preferred_element_type=jnp.float32)
    residual = residual + b3_ref[...]
    residual = jnp.where(any_valid, residual, 0.0)

    fine = coarse + residual
    fine_ref[...] = fine
    warped_ref[...] = pc0 + fine


@jax.jit
def kernel(pc0_points, pc1_points, coarse_flow, W1, b1, W2, b2, W3, b3):
    pc1t = jnp.concatenate([pc1_points.T, jnp.zeros((5, M), jnp.float32)], axis=0)
    grid = (N // BLK,)
    row_spec = pl.BlockSpec((BLK, 3), lambda i: (i, 0))
    full = lambda shape: pl.BlockSpec(shape, lambda i: tuple(0 for _ in shape))
    fine, warped = pl.pallas_call(
        _knn_mlp_kernel,
        grid=grid,
        in_specs=[
            row_spec,
            row_spec,
            full((M, 3)),
            full((8, M)),
            full((6, HIDDEN)),
            full((1, HIDDEN)),
            full((HIDDEN, HIDDEN)),
            full((1, HIDDEN)),
            full((HIDDEN, 3)),
            full((1, 3)),
        ],
        out_specs=[row_spec, row_spec],
        out_shape=[
            jax.ShapeDtypeStruct((N, 3), jnp.float32),
            jax.ShapeDtypeStruct((N, 3), jnp.float32),
        ],
    )(pc0_points, coarse_flow, pc1_points, pc1t,
      W1, b1.reshape(1, HIDDEN), W2, b2.reshape(1, HIDDEN),
      W3, b3.reshape(1, 3))
    return fine, warped


# speedup vs baseline: 5.2512x; 1.0967x over previous
import jax
import jax.numpy as jnp
from jax.experimental import pallas as pl
from jax.experimental.pallas import tpu as pltpu

N = 8192
M = 8192
K = 16
HIDDEN = 64
BLK = 256
STOP = 1.000001


def _knn_mlp_kernel(pc0_ref, coarse_ref, pc1_ref, pc1t_ref,
                    w1_ref, b1_ref, w2_ref, b2_ref, w3_ref, b3_ref,
                    fine_ref, warped_ref,
                    d2_ref, pooled_ref, anyv_ref, cont_ref):
    pc0 = pc0_ref[...]
    coarse = coarse_ref[...]
    target = pc0 + coarse
    pc1 = pc1_ref[...]
    pc1t = pc1t_ref[...]

    tn = jnp.sum(target * target, axis=1, keepdims=True)
    pn = jnp.sum(pc1t * pc1t, axis=0, keepdims=True)
    tpad = jnp.concatenate(
        [target, jnp.zeros((BLK, 5), jnp.float32)], axis=1)
    dot = jnp.dot(tpad.astype(jnp.bfloat16), pc1t.astype(jnp.bfloat16),
                  preferred_element_type=jnp.float32)
    d2_ref[...] = tn + pn - 2.0 * dot
    pooled_ref[...] = jnp.full((BLK, HIDDEN), -1e9, jnp.float32)
    anyv_ref[...] = jnp.zeros((BLK, 1), jnp.float32)
    cont_ref[0, 0] = jnp.int32(1)

    col = jax.lax.broadcasted_iota(jnp.int32, (BLK, M), 1)
    w1 = w1_ref[...]
    b1 = b1_ref[...]
    w2 = w2_ref[...]
    b2 = b2_ref[...]

    for _ in range(K):
        @pl.when(cont_ref[0, 0] == 1)
        def _():
            d2 = d2_ref[...]
            m = jnp.min(d2, axis=1, keepdims=True)
            is_min = d2 == m
            idx = jnp.min(jnp.where(is_min, col, jnp.int32(M)),
                          axis=1, keepdims=True)
            sel = col == idx
            onehot = sel.astype(jnp.float32)
            pts = jnp.dot(onehot, pc1, preferred_element_type=jnp.float32)
            valid = jnp.sqrt(jnp.maximum(m, 1e-12)) < 1.0
            feats = jnp.concatenate([pts - target, pc0], axis=1)
            h = jnp.dot(feats, w1, preferred_element_type=jnp.float32)
            h = jnp.maximum(h + b1, 0.0)
            h = jnp.dot(h, w2, preferred_element_type=jnp.float32)
            h = jnp.maximum(h + b2, 0.0)
            pooled = pooled_ref[...]
            pooled_ref[...] = jnp.where(valid, jnp.maximum(pooled, h), pooled)
            anyv = anyv_ref[...]
            anyv_ref[...] = jnp.maximum(anyv, valid.astype(jnp.float32))
            d2_ref[...] = jnp.where(sel, jnp.float32(3e38), d2)
            cont_ref[0, 0] = (jnp.min(m) < STOP).astype(jnp.int32)

    any_valid = anyv_ref[...] > 0.5
    pooled = jnp.where(any_valid, pooled_ref[...], 0.0)
    residual = jnp.dot(pooled, w3_ref[...], preferred_element_type=jnp.float32)
    residual = residual + b3_ref[...]
    residual = jnp.where(any_valid, residual, 0.0)

    fine = coarse + residual
    fine_ref[...] = fine
    warped_ref[...] = pc0 + fine


@jax.jit
def kernel(pc0_points, pc1_points, coarse_flow, W1, b1, W2, b2, W3, b3):
    pc1t = jnp.concatenate([pc1_points.T, jnp.zeros((5, M), jnp.float32)], axis=0)
    grid = (N // BLK,)
    row_spec = pl.BlockSpec((BLK, 3), lambda i: (i, 0))
    full = lambda shape: pl.BlockSpec(shape, lambda i: tuple(0 for _ in shape))
    fine, warped = pl.pallas_call(
        _knn_mlp_kernel,
        grid=grid,
        in_specs=[
            row_spec,
            row_spec,
            full((M, 3)),
            full((8, M)),
            full((6, HIDDEN)),
            full((1, HIDDEN)),
            full((HIDDEN, HIDDEN)),
            full((1, HIDDEN)),
            full((HIDDEN, 3)),
            full((1, 3)),
        ],
        out_specs=[row_spec, row_spec],
        out_shape=[
            jax.ShapeDtypeStruct((N, 3), jnp.float32),
            jax.ShapeDtypeStruct((N, 3), jnp.float32),
        ],
        scratch_shapes=[
            pltpu.VMEM((BLK, M), jnp.float32),
            pltpu.VMEM((BLK, HIDDEN), jnp.float32),
            pltpu.VMEM((BLK, 1), jnp.float32),
            pltpu.SMEM((1, 1), jnp.int32),
        ],
    )(pc0_points, coarse_flow, pc1_points, pc1t,
      W1, b1.reshape(1, HIDDEN), W2, b2.reshape(1, HIDDEN),
      W3, b3.reshape(1, 3))
    return fine, warped


# speedup vs baseline: 6.4490x; 1.2281x over previous
import jax
import jax.numpy as jnp
from jax.experimental import pallas as pl
from jax.experimental.pallas import tpu as pltpu

N = 8192
M = 8192
K = 16
HIDDEN = 64
BLK = 256
VT = 0x3F800000
MASKED = 0x7FFFFFFF


def _knn_mlp_kernel(pc0_ref, coarse_ref, pc1_ref, pc1t_ref,
                    w1_ref, b1_ref, w2_ref, b2_ref, w3_ref, b3_ref,
                    fine_ref, warped_ref,
                    p_ref, pooled_ref, anyv_ref, cont_ref):
    pc0 = pc0_ref[...]
    coarse = coarse_ref[...]
    target = pc0 + coarse
    pc1 = pc1_ref[...]
    pc1t = pc1t_ref[...]

    tn = jnp.sum(target * target, axis=1, keepdims=True)
    pn = jnp.sum(pc1t * pc1t, axis=0, keepdims=True)
    tpad = jnp.concatenate(
        [target, jnp.zeros((BLK, 5), jnp.float32)], axis=1)
    dot = jnp.dot(tpad.astype(jnp.bfloat16), pc1t.astype(jnp.bfloat16),
                  preferred_element_type=jnp.float32)
    d2 = jnp.maximum(tn + pn - 2.0 * dot, 0.0)

    col = jax.lax.broadcasted_iota(jnp.int32, (BLK, M), 1)
    bits = jax.lax.bitcast_convert_type(d2, jnp.int32)
    p_ref[...] = jnp.bitwise_or(
        jnp.bitwise_and(bits, jnp.int32(~0x1FFF)), col)
    pooled_ref[...] = jnp.full((BLK, HIDDEN), -1e9, jnp.float32)
    anyv_ref[...] = jnp.zeros((BLK, 1), jnp.float32)
    cont_ref[0, 0] = jnp.int32(1)

    w1 = w1_ref[...]
    b1 = b1_ref[...]
    w2 = w2_ref[...]
    b2 = b2_ref[...]

    for _ in range(K):
        @pl.when(cont_ref[0, 0] == 1)
        def _():
            p = p_ref[...]
            mp = jnp.min(p, axis=1, keepdims=True)
            sel = p == mp
            onehot = sel.astype(jnp.float32)
            pts = jnp.dot(onehot, pc1, preferred_element_type=jnp.float32)
            valid = mp < VT
            feats = jnp.concatenate([pts - target, pc0], axis=1)
            h = jnp.dot(feats, w1, preferred_element_type=jnp.float32)
            h = jnp.maximum(h + b1, 0.0)
            h = jnp.dot(h, w2, preferred_element_type=jnp.float32)
            h = jnp.maximum(h + b2, 0.0)
            pooled = pooled_ref[...]
            pooled_ref[...] = jnp.where(valid, jnp.maximum(pooled, h), pooled)
            anyv = anyv_ref[...]
            anyv_ref[...] = jnp.maximum(anyv, valid.astype(jnp.float32))
            p_ref[...] = jnp.where(sel, jnp.int32(MASKED), p)
            cont_ref[0, 0] = (jnp.min(mp) < VT).astype(jnp.int32)

    any_valid = anyv_ref[...] > 0.5
    pooled = jnp.where(any_valid, pooled_ref[...], 0.0)
    residual = jnp.dot(pooled, w3_ref[...], preferred_element_type=jnp.float32)
    residual = residual + b3_ref[...]
    residual = jnp.where(any_valid, residual, 0.0)

    fine = coarse + residual
    fine_ref[...] = fine
    warped_ref[...] = pc0 + fine


@jax.jit
def kernel(pc0_points, pc1_points, coarse_flow, W1, b1, W2, b2, W3, b3):
    pc1t = jnp.concatenate([pc1_points.T, jnp.zeros((5, M), jnp.float32)], axis=0)
    grid = (N // BLK,)
    row_spec = pl.BlockSpec((BLK, 3), lambda i: (i, 0))
    full = lambda shape: pl.BlockSpec(shape, lambda i: tuple(0 for _ in shape))
    fine, warped = pl.pallas_call(
        _knn_mlp_kernel,
        grid=grid,
        in_specs=[
            row_spec,
            row_spec,
            full((M, 3)),
            full((8, M)),
            full((6, HIDDEN)),
            full((1, HIDDEN)),
            full((HIDDEN, HIDDEN)),
            full((1, HIDDEN)),
            full((HIDDEN, 3)),
            full((1, 3)),
        ],
        out_specs=[row_spec, row_spec],
        out_shape=[
            jax.ShapeDtypeStruct((N, 3), jnp.float32),
            jax.ShapeDtypeStruct((N, 3), jnp.float32),
        ],
        scratch_shapes=[
            pltpu.VMEM((BLK, M), jnp.int32),
            pltpu.VMEM((BLK, HIDDEN), jnp.float32),
            pltpu.VMEM((BLK, 1), jnp.float32),
            pltpu.SMEM((1, 1), jnp.int32),
        ],
    )(pc0_points, coarse_flow, pc1_points, pc1t,
      W1, b1.reshape(1, HIDDEN), W2, b2.reshape(1, HIDDEN),
      W3, b3.reshape(1, 3))
    return fine, warped


# speedup vs baseline: 6.8965x; 1.0694x over previous
import jax
import jax.numpy as jnp
from jax.experimental import pallas as pl
from jax.experimental.pallas import tpu as pltpu

N = 8192
M = 8192
K = 16
HIDDEN = 64
BLK = 256
VT = 0x3F800000
MASKED = 0x7FFFFFFF


def _knn_mlp_kernel(pc0_ref, coarse_ref, pc1_ref, pc1t_ref,
                    w1_ref, b1_ref, w2_ref, b2_ref, w3_ref, b3_ref,
                    fine_ref, warped_ref,
                    p_ref, pooled_ref, anyv_ref, prev_ref, cont_ref):
    pc0 = pc0_ref[...]
    coarse = coarse_ref[...]
    target = pc0 + coarse
    pc1 = pc1_ref[...]
    pc1t = pc1t_ref[...]

    tn = jnp.sum(target * target, axis=1, keepdims=True)
    pn = jnp.sum(pc1t * pc1t, axis=0, keepdims=True)
    tpad = jnp.concatenate(
        [target, jnp.zeros((BLK, 5), jnp.float32)], axis=1)
    dot = jnp.dot(tpad.astype(jnp.bfloat16), pc1t.astype(jnp.bfloat16),
                  preferred_element_type=jnp.float32)
    d2 = jnp.maximum(tn + pn - 2.0 * dot, 0.0)

    col = jax.lax.broadcasted_iota(jnp.int32, (BLK, M), 1)
    bits = jax.lax.bitcast_convert_type(d2, jnp.int32)
    p_ref[...] = jnp.bitwise_or(
        jnp.bitwise_and(bits, jnp.int32(~0x1FFF)), col)
    pooled_ref[...] = jnp.full((BLK, HIDDEN), -1e9, jnp.float32)
    anyv_ref[...] = jnp.zeros((BLK, 1), jnp.float32)
    cont_ref[0, 0] = jnp.int32(1)

    w1 = w1_ref[...]
    b1 = b1_ref[...]
    w2 = w2_ref[...]
    b2 = b2_ref[...]

    prev_ref[...] = jnp.full((BLK, 1), -1, jnp.int32)

    for _ in range(K):
        @pl.when(cont_ref[0, 0] == 1)
        def _():
            p = p_ref[...]
            prev = prev_ref[...]
            mp = jnp.min(jnp.where(p > prev, p, jnp.int32(MASKED)),
                         axis=1, keepdims=True)
            sel = p == mp
            onehot = sel.astype(jnp.float32)
            pts = jnp.dot(onehot, pc1, preferred_element_type=jnp.float32)
            valid = mp < VT
            feats = jnp.concatenate([pts - target, pc0], axis=1)
            h = jnp.dot(feats, w1, preferred_element_type=jnp.float32)
            h = jnp.maximum(h + b1, 0.0)
            h = jnp.dot(h, w2, preferred_element_type=jnp.float32)
            h = jnp.maximum(h + b2, 0.0)
            pooled = pooled_ref[...]
            pooled_ref[...] = jnp.where(valid, jnp.maximum(pooled, h), pooled)
            anyv = anyv_ref[...]
            anyv_ref[...] = jnp.maximum(anyv, valid.astype(jnp.float32))
            prev_ref[...] = mp
            cont_ref[0, 0] = (jnp.min(mp) < VT).astype(jnp.int32)

    any_valid = anyv_ref[...] > 0.5
    pooled = jnp.where(any_valid, pooled_ref[...], 0.0)
    residual = jnp.dot(pooled, w3_ref[...], preferred_element_type=jnp.float32)
    residual = residual + b3_ref[...]
    residual = jnp.where(any_valid, residual, 0.0)

    fine = coarse + residual
    fine_ref[...] = fine
    warped_ref[...] = pc0 + fine


@jax.jit
def kernel(pc0_points, pc1_points, coarse_flow, W1, b1, W2, b2, W3, b3):
    pc1t = jnp.concatenate([pc1_points.T, jnp.zeros((5, M), jnp.float32)], axis=0)
    grid = (N // BLK,)
    row_spec = pl.BlockSpec((BLK, 3), lambda i: (i, 0))
    full = lambda shape: pl.BlockSpec(shape, lambda i: tuple(0 for _ in shape))
    fine, warped = pl.pallas_call(
        _knn_mlp_kernel,
        grid=grid,
        in_specs=[
            row_spec,
            row_spec,
            full((M, 3)),
            full((8, M)),
            full((6, HIDDEN)),
            full((1, HIDDEN)),
            full((HIDDEN, HIDDEN)),
            full((1, HIDDEN)),
            full((HIDDEN, 3)),
            full((1, 3)),
        ],
        out_specs=[row_spec, row_spec],
        out_shape=[
            jax.ShapeDtypeStruct((N, 3), jnp.float32),
            jax.ShapeDtypeStruct((N, 3), jnp.float32),
        ],
        scratch_shapes=[
            pltpu.VMEM((BLK, M), jnp.int32),
            pltpu.VMEM((BLK, HIDDEN), jnp.float32),
            pltpu.VMEM((BLK, 1), jnp.float32),
            pltpu.VMEM((BLK, 1), jnp.int32),
            pltpu.SMEM((1, 1), jnp.int32),
        ],
    )(pc0_points, coarse_flow, pc1_points, pc1t,
      W1, b1.reshape(1, HIDDEN), W2, b2.reshape(1, HIDDEN),
      W3, b3.reshape(1, 3))
    return fine, warped


# speedup vs baseline: 10.3234x; 1.4969x over previous
import jax
import jax.numpy as jnp
from jax.experimental import pallas as pl
from jax.experimental.pallas import tpu as pltpu

N = 8192
M = 8192
K = 16
HIDDEN = 64
BLK = 128
VT = 0x3F800000
MASKED = 0x7FFFFFFF


def _knn_mlp_kernel(pc0_ref, coarse_ref, pc1_ref, pc1t_ref,
                    w1_ref, b1_ref, w2_ref, b2_ref, w3_ref, b3_ref,
                    fine_ref, warped_ref,
                    p_ref, pooled_ref, anyv_ref, prev_ref, cont_ref):
    pc0 = pc0_ref[...]
    coarse = coarse_ref[...]
    target = pc0 + coarse
    pc1 = pc1_ref[...]
    pc1t = pc1t_ref[...]

    tn = jnp.sum(target * target, axis=1, keepdims=True)
    pn = jnp.sum(pc1t * pc1t, axis=0, keepdims=True)
    tpad = jnp.concatenate(
        [target, jnp.zeros((BLK, 5), jnp.float32)], axis=1)
    dot = jnp.dot(tpad.astype(jnp.bfloat16), pc1t.astype(jnp.bfloat16),
                  preferred_element_type=jnp.float32)
    d2 = jnp.maximum(tn + pn - 2.0 * dot, 0.0)

    col = jax.lax.broadcasted_iota(jnp.int32, (BLK, M), 1)
    bits = jax.lax.bitcast_convert_type(d2, jnp.int32)
    p_ref[...] = jnp.bitwise_or(
        jnp.bitwise_and(bits, jnp.int32(~0x1FFF)), col)
    pooled_ref[...] = jnp.full((BLK, HIDDEN), -1e9, jnp.float32)
    anyv_ref[...] = jnp.zeros((BLK, 1), jnp.float32)
    cont_ref[0, 0] = jnp.int32(1)

    w1 = w1_ref[...]
    b1 = b1_ref[...]
    w2 = w2_ref[...]
    b2 = b2_ref[...]

    prev_ref[...] = jnp.full((BLK, 1), -1, jnp.int32)

    for _ in range(K):
        @pl.when(cont_ref[0, 0] == 1)
        def _():
            p = p_ref[...]
            prev = prev_ref[...]
            mp = jnp.min(jnp.where(p > prev, p, jnp.int32(MASKED)),
                         axis=1, keepdims=True)
            sel = p == mp
            onehot = sel.astype(jnp.float32)
            pts = jnp.dot(onehot, pc1, preferred_element_type=jnp.float32)
            valid = mp < VT
            feats = jnp.concatenate([pts - target, pc0], axis=1)
            h = jnp.dot(feats, w1, preferred_element_type=jnp.float32)
            h = jnp.maximum(h + b1, 0.0)
            h = jnp.dot(h, w2, preferred_element_type=jnp.float32)
            h = jnp.maximum(h + b2, 0.0)
            pooled = pooled_ref[...]
            pooled_ref[...] = jnp.where(valid, jnp.maximum(pooled, h), pooled)
            anyv = anyv_ref[...]
            anyv_ref[...] = jnp.maximum(anyv, valid.astype(jnp.float32))
            prev_ref[...] = mp
            cont_ref[0, 0] = (jnp.min(mp) < VT).astype(jnp.int32)

    any_valid = anyv_ref[...] > 0.5
    pooled = jnp.where(any_valid, pooled_ref[...], 0.0)
    residual = jnp.dot(pooled, w3_ref[...], preferred_element_type=jnp.float32)
    residual = residual + b3_ref[...]
    residual = jnp.where(any_valid, residual, 0.0)

    fine = coarse + residual
    fine_ref[...] = fine
    warped_ref[...] = pc0 + fine


@jax.jit
def kernel(pc0_points, pc1_points, coarse_flow, W1, b1, W2, b2, W3, b3):
    pc1t = jnp.concatenate([pc1_points.T, jnp.zeros((5, M), jnp.float32)], axis=0)
    grid = (N // BLK,)
    row_spec = pl.BlockSpec((BLK, 3), lambda i: (i, 0))
    full = lambda shape: pl.BlockSpec(shape, lambda i: tuple(0 for _ in shape))
    fine, warped = pl.pallas_call(
        _knn_mlp_kernel,
        grid=grid,
        in_specs=[
            row_spec,
            row_spec,
            full((M, 3)),
            full((8, M)),
            full((6, HIDDEN)),
            full((1, HIDDEN)),
            full((HIDDEN, HIDDEN)),
            full((1, HIDDEN)),
            full((HIDDEN, 3)),
            full((1, 3)),
        ],
        out_specs=[row_spec, row_spec],
        out_shape=[
            jax.ShapeDtypeStruct((N, 3), jnp.float32),
            jax.ShapeDtypeStruct((N, 3), jnp.float32),
        ],
        scratch_shapes=[
            pltpu.VMEM((BLK, M), jnp.int32),
            pltpu.VMEM((BLK, HIDDEN), jnp.float32),
            pltpu.VMEM((BLK, 1), jnp.float32),
            pltpu.VMEM((BLK, 1), jnp.int32),
            pltpu.SMEM((1, 1), jnp.int32),
        ],
    )(pc0_points, coarse_flow, pc1_points, pc1t,
      W1, b1.reshape(1, HIDDEN), W2, b2.reshape(1, HIDDEN),
      W3, b3.reshape(1, 3))
    return fine, warped
